# baseline (speedup 1.0000x reference)
import functools

import jax
import jax.numpy as jnp
from jax.experimental import pallas as pl
from jax.experimental.pallas import tpu as pltpu

N = 4096
D = 128
H = 256

_HI = jax.lax.Precision.DEFAULT

_A_BLK = 512
_B_I = 1024
_B_K = 256
_B_EPI = 512


def _pass_a_kernel(x_ref, w1_ref, b1_ref, adj_ref, w2_ref, z_ref, s_ref):
    @pl.when(pl.program_id(0) == 0)
    def _():
        s_ref[...] = jnp.dot(x_ref[...], w1_ref[...], precision=_HI)

    h = jnp.dot(adj_ref[...], s_ref[...], precision=_HI) + b1_ref[...]
    h = jnp.where(h >= 0, h, 0.01 * h)
    z_ref[...] = jnp.dot(h, w2_ref[...], precision=_HI)


def _pass_b_kernel(adj_ref, z_ref, b2_ref, mask_hbm_ref, out_hbm_ref,
                   acc_ref, mask_ref, in_sem, out_sem):
    i = pl.program_id(0)
    k = pl.program_id(1)
    nk = pl.num_programs(1)

    mask_copy = pltpu.make_async_copy(
        mask_hbm_ref.at[pl.ds(i * _B_I, _B_I), :], mask_ref, in_sem
    )

    @pl.when(k == 0)
    def _():
        mask_copy.start()

    a = adj_ref[...]

    @pl.when(k == 0)
    def _():
        for j in range(N // _B_EPI):
            sl = pl.ds(j * _B_EPI, _B_EPI)
            acc_ref[:, sl] = jnp.dot(a, z_ref[:, sl], precision=_HI)

    @pl.when((k > 0) & (k < nk - 1))
    def _():
        for j in range(N // _B_EPI):
            sl = pl.ds(j * _B_EPI, _B_EPI)
            acc_ref[:, sl] += jnp.dot(a, z_ref[:, sl], precision=_HI)

    @pl.when(k == nk - 1)
    def _():
        c = _B_EPI
        m = jnp.full((_B_I, 1), -jnp.inf, dtype=jnp.float32)
        for j in range(N // c):
            sl = pl.ds(j * c, c)
            t = (acc_ref[:, sl] + jnp.dot(a, z_ref[:, sl], precision=_HI)
                 + b2_ref[:, sl])
            acc_ref[:, sl] = t
            m = jnp.maximum(m, jnp.max(t, axis=1, keepdims=True))
        denom = jnp.zeros((_B_I, 1), dtype=jnp.float32)
        for j in range(N // c):
            sl = pl.ds(j * c, c)
            t = jnp.exp(acc_ref[:, sl] - m)
            acc_ref[:, sl] = t
            denom = denom + jnp.sum(t, axis=1, keepdims=True)
        r = 1.0 / denom
        mask_copy.wait()
        out_copies = []
        for j in range(N // c):
            sl = pl.ds(j * c, c)
            acc_ref[:, sl] = acc_ref[:, sl] * r * mask_ref[:, sl].astype(jnp.float32)
            cp = pltpu.make_async_copy(
                acc_ref.at[:, sl],
                out_hbm_ref.at[pl.ds(i * _B_I, _B_I), sl],
                out_sem,
            )
            cp.start()
            out_copies.append(cp)
        for cp in out_copies:
            cp.wait()


@functools.partial(jax.jit, static_argnames=())
def _run(x, adj, edge_mask, W1, b1, W2, b2):
    b1r = b1.reshape(1, H)
    b2r = b2.reshape(1, N)

    z = pl.pallas_call(
        _pass_a_kernel,
        grid=(N // _A_BLK,),
        in_specs=[
            pl.BlockSpec((N, D), lambda i: (0, 0)),
            pl.BlockSpec((D, H), lambda i: (0, 0)),
            pl.BlockSpec((1, H), lambda i: (0, 0)),
            pl.BlockSpec((_A_BLK, N), lambda i: (i, 0)),
            pl.BlockSpec((H, N), lambda i: (0, 0)),
        ],
        out_specs=pl.BlockSpec((_A_BLK, N), lambda i: (i, 0)),
        out_shape=jax.ShapeDtypeStruct((N, N), jnp.float32),
        scratch_shapes=[pltpu.VMEM((N, H), jnp.float32)],
        compiler_params=pltpu.CompilerParams(
            dimension_semantics=("arbitrary",),
        ),
    )(x, W1, b1r, adj, W2)

    out = pl.pallas_call(
        _pass_b_kernel,
        grid=(N // _B_I, N // _B_K),
        in_specs=[
            pl.BlockSpec((_B_I, _B_K), lambda i, k: (i, k)),
            pl.BlockSpec((_B_K, N), lambda i, k: (k, 0)),
            pl.BlockSpec((1, N), lambda i, k: (0, 0)),
            pl.BlockSpec(memory_space=pltpu.MemorySpace.HBM),
        ],
        out_specs=pl.BlockSpec(memory_space=pltpu.MemorySpace.HBM),
        out_shape=jax.ShapeDtypeStruct((N, N), jnp.float32),
        scratch_shapes=[
            pltpu.VMEM((_B_I, N), jnp.float32),
            pltpu.VMEM((_B_I, N), jnp.bfloat16),
            pltpu.SemaphoreType.DMA,
            pltpu.SemaphoreType.DMA,
        ],
        compiler_params=pltpu.CompilerParams(
            dimension_semantics=("arbitrary", "arbitrary"),
        ),
    )(adj, z, b2r, edge_mask.astype(jnp.bfloat16))

    return out


def kernel(x, adj, edge_mask, W1, b1, W2, b2, dropout):
    return _run(x, adj, edge_mask, W1, b1, W2, b2)


# speedup vs baseline: 1.0187x; 1.0187x over previous
import functools

import jax
import jax.numpy as jnp
from jax.experimental import pallas as pl
from jax.experimental.pallas import tpu as pltpu

N = 4096
D = 128
H = 256

_HI = jax.lax.Precision.DEFAULT

_A_BLK = 512
_B_I = 1024
_B_K = 512
_B_EPI = 512


def _pass_a_kernel(x_ref, w1_ref, b1_ref, adj_ref, w2_ref, z_ref, s_ref):
    @pl.when(pl.program_id(0) == 0)
    def _():
        s_ref[...] = jnp.dot(x_ref[...], w1_ref[...], precision=_HI)

    h = jnp.dot(adj_ref[...], s_ref[...], precision=_HI) + b1_ref[...]
    h = jnp.where(h >= 0, h, 0.01 * h)
    z_ref[...] = jnp.dot(h, w2_ref[...], precision=_HI)


def _pass_b_kernel(adj_ref, z_ref, b2_ref, mask_hbm_ref, out_hbm_ref,
                   acc_ref, mask_ref, in_sem, out_sem):
    i = pl.program_id(0)
    k = pl.program_id(1)
    nk = pl.num_programs(1)


    a = adj_ref[...]

    @pl.when(k == 0)
    def _():
        for j in range(N // _B_EPI):
            sl = pl.ds(j * _B_EPI, _B_EPI)
            acc_ref[:, sl] = jnp.dot(a, z_ref[:, sl], precision=_HI)

    @pl.when((k > 0) & (k < nk - 1))
    def _():
        for j in range(N // _B_EPI):
            sl = pl.ds(j * _B_EPI, _B_EPI)
            acc_ref[:, sl] += jnp.dot(a, z_ref[:, sl], precision=_HI)

    @pl.when(k == nk - 1)
    def _():
        c = _B_EPI
        m = jnp.full((_B_I, 1), -jnp.inf, dtype=jnp.float32)
        for j in range(N // c):
            sl = pl.ds(j * c, c)
            t = (acc_ref[:, sl] + jnp.dot(a, z_ref[:, sl], precision=_HI)
                 + b2_ref[:, sl])
            acc_ref[:, sl] = t
            m = jnp.maximum(m, jnp.max(t, axis=1, keepdims=True))
        nchunks = N // c

        def _mask_copy(j, buf):
            return pltpu.make_async_copy(
                mask_hbm_ref.at[pl.ds(i * _B_I, _B_I), pl.ds(j * c, c)],
                mask_ref.at[buf], in_sem.at[buf],
            )

        _mask_copy(0, 0).start()
        _mask_copy(1, 1).start()
        denom = jnp.zeros((_B_I, 1), dtype=jnp.float32)
        for j in range(nchunks):
            sl = pl.ds(j * c, c)
            t = jnp.exp(acc_ref[:, sl] - m)
            denom = denom + jnp.sum(t, axis=1, keepdims=True)
            _mask_copy(j, j % 2).wait()
            acc_ref[:, sl] = t * mask_ref[j % 2].astype(jnp.float32)
            if j + 2 < nchunks:
                _mask_copy(j + 2, j % 2).start()
        r = 1.0 / denom
        out_copies = []
        for j in range(nchunks):
            sl = pl.ds(j * c, c)
            acc_ref[:, sl] = acc_ref[:, sl] * r
            cp = pltpu.make_async_copy(
                acc_ref.at[:, sl],
                out_hbm_ref.at[pl.ds(i * _B_I, _B_I), sl],
                out_sem,
            )
            cp.start()
            out_copies.append(cp)
        for cp in out_copies:
            cp.wait()


@functools.partial(jax.jit, static_argnames=())
def _run(x, adj, edge_mask, W1, b1, W2, b2):
    b1r = b1.reshape(1, H)
    b2r = b2.reshape(1, N)

    z = pl.pallas_call(
        _pass_a_kernel,
        grid=(N // _A_BLK,),
        in_specs=[
            pl.BlockSpec((N, D), lambda i: (0, 0)),
            pl.BlockSpec((D, H), lambda i: (0, 0)),
            pl.BlockSpec((1, H), lambda i: (0, 0)),
            pl.BlockSpec((_A_BLK, N), lambda i: (i, 0)),
            pl.BlockSpec((H, N), lambda i: (0, 0)),
        ],
        out_specs=pl.BlockSpec((_A_BLK, N), lambda i: (i, 0)),
        out_shape=jax.ShapeDtypeStruct((N, N), jnp.float32),
        scratch_shapes=[pltpu.VMEM((N, H), jnp.float32)],
        compiler_params=pltpu.CompilerParams(
            dimension_semantics=("arbitrary",),
        ),
    )(x, W1, b1r, adj, W2)

    out = pl.pallas_call(
        _pass_b_kernel,
        grid=(N // _B_I, N // _B_K),
        in_specs=[
            pl.BlockSpec((_B_I, _B_K), lambda i, k: (i, k)),
            pl.BlockSpec((_B_K, N), lambda i, k: (k, 0)),
            pl.BlockSpec((1, N), lambda i, k: (0, 0)),
            pl.BlockSpec(memory_space=pltpu.MemorySpace.HBM),
        ],
        out_specs=pl.BlockSpec(memory_space=pltpu.MemorySpace.HBM),
        out_shape=jax.ShapeDtypeStruct((N, N), jnp.float32),
        scratch_shapes=[
            pltpu.VMEM((_B_I, N), jnp.float32),
            pltpu.VMEM((2, _B_I, _B_EPI), jnp.bfloat16),
            pltpu.SemaphoreType.DMA((2,)),
            pltpu.SemaphoreType.DMA,
        ],
        compiler_params=pltpu.CompilerParams(
            dimension_semantics=("arbitrary", "arbitrary"),
        ),
    )(adj, z, b2r, edge_mask.astype(jnp.bfloat16))

    return out


def kernel(x, adj, edge_mask, W1, b1, W2, b2, dropout):
    return _run(x, adj, edge_mask, W1, b1, W2, b2)


# speedup vs baseline: 1.0289x; 1.0100x over previous
import functools

import jax
import jax.numpy as jnp
from jax.experimental import pallas as pl
from jax.experimental.pallas import tpu as pltpu

N = 4096
D = 128
H = 256

_HI = jax.lax.Precision.DEFAULT

_A_BLK = 512
_B_I = 2048
_B_K = 256
_B_EPI = 256


def _pass_a_kernel(x_ref, w1_ref, b1_ref, adj_ref, w2_ref, z_ref, s_ref):
    @pl.when(pl.program_id(0) == 0)
    def _():
        s_ref[...] = jnp.dot(x_ref[...], w1_ref[...], precision=_HI)

    h = jnp.dot(adj_ref[...], s_ref[...], precision=_HI) + b1_ref[...]
    h = jnp.where(h >= 0, h, 0.01 * h)
    z_ref[...] = jnp.dot(h, w2_ref[...], precision=_HI)


def _pass_b_kernel(adj_ref, z_ref, b2_ref, mask_hbm_ref, out_hbm_ref,
                   acc_ref, mask_ref, in_sem, out_sem):
    i = pl.program_id(0)
    k = pl.program_id(1)
    nk = pl.num_programs(1)


    a = adj_ref[...]

    @pl.when(k == 0)
    def _():
        for j in range(N // _B_EPI):
            sl = pl.ds(j * _B_EPI, _B_EPI)
            acc_ref[:, sl] = jnp.dot(a, z_ref[:, sl], precision=_HI)

    @pl.when((k > 0) & (k < nk - 1))
    def _():
        for j in range(N // _B_EPI):
            sl = pl.ds(j * _B_EPI, _B_EPI)
            acc_ref[:, sl] += jnp.dot(a, z_ref[:, sl], precision=_HI)

    @pl.when(k == nk - 1)
    def _():
        c = _B_EPI
        m = jnp.full((_B_I, 1), -jnp.inf, dtype=jnp.float32)
        for j in range(N // c):
            sl = pl.ds(j * c, c)
            t = (acc_ref[:, sl] + jnp.dot(a, z_ref[:, sl], precision=_HI)
                 + b2_ref[:, sl])
            acc_ref[:, sl] = t
            m = jnp.maximum(m, jnp.max(t, axis=1, keepdims=True))
        nchunks = N // c

        def _mask_copy(j, buf):
            return pltpu.make_async_copy(
                mask_hbm_ref.at[pl.ds(i * _B_I, _B_I), pl.ds(j * c, c)],
                mask_ref.at[buf], in_sem.at[buf],
            )

        _mask_copy(0, 0).start()
        _mask_copy(1, 1).start()
        denom = jnp.zeros((_B_I, 1), dtype=jnp.float32)
        for j in range(nchunks):
            sl = pl.ds(j * c, c)
            t = jnp.exp(acc_ref[:, sl] - m)
            denom = denom + jnp.sum(t, axis=1, keepdims=True)
            _mask_copy(j, j % 2).wait()
            acc_ref[:, sl] = t * mask_ref[j % 2].astype(jnp.float32)
            if j + 2 < nchunks:
                _mask_copy(j + 2, j % 2).start()
        r = 1.0 / denom
        out_copies = []
        for j in range(nchunks):
            sl = pl.ds(j * c, c)
            acc_ref[:, sl] = acc_ref[:, sl] * r
            cp = pltpu.make_async_copy(
                acc_ref.at[:, sl],
                out_hbm_ref.at[pl.ds(i * _B_I, _B_I), sl],
                out_sem,
            )
            cp.start()
            out_copies.append(cp)
        for cp in out_copies:
            cp.wait()


@functools.partial(jax.jit, static_argnames=())
def _run(x, adj, edge_mask, W1, b1, W2, b2):
    b1r = b1.reshape(1, H)
    b2r = b2.reshape(1, N)

    z = pl.pallas_call(
        _pass_a_kernel,
        grid=(N // _A_BLK,),
        in_specs=[
            pl.BlockSpec((N, D), lambda i: (0, 0)),
            pl.BlockSpec((D, H), lambda i: (0, 0)),
            pl.BlockSpec((1, H), lambda i: (0, 0)),
            pl.BlockSpec((_A_BLK, N), lambda i: (i, 0)),
            pl.BlockSpec((H, N), lambda i: (0, 0)),
        ],
        out_specs=pl.BlockSpec((_A_BLK, N), lambda i: (i, 0)),
        out_shape=jax.ShapeDtypeStruct((N, N), jnp.float32),
        scratch_shapes=[pltpu.VMEM((N, H), jnp.float32)],
        compiler_params=pltpu.CompilerParams(
            dimension_semantics=("arbitrary",),
        ),
    )(x, W1, b1r, adj, W2)

    out = pl.pallas_call(
        _pass_b_kernel,
        grid=(N // _B_I, N // _B_K),
        in_specs=[
            pl.BlockSpec((_B_I, _B_K), lambda i, k: (i, k)),
            pl.BlockSpec((_B_K, N), lambda i, k: (k, 0)),
            pl.BlockSpec((1, N), lambda i, k: (0, 0)),
            pl.BlockSpec(memory_space=pltpu.MemorySpace.HBM),
        ],
        out_specs=pl.BlockSpec(memory_space=pltpu.MemorySpace.HBM),
        out_shape=jax.ShapeDtypeStruct((N, N), jnp.float32),
        scratch_shapes=[
            pltpu.VMEM((_B_I, N), jnp.float32),
            pltpu.VMEM((2, _B_I, _B_EPI), jnp.bfloat16),
            pltpu.SemaphoreType.DMA((2,)),
            pltpu.SemaphoreType.DMA,
        ],
        compiler_params=pltpu.CompilerParams(
            dimension_semantics=("arbitrary", "arbitrary"),
        ),
    )(adj, z, b2r, edge_mask.astype(jnp.bfloat16))

    return out


def kernel(x, adj, edge_mask, W1, b1, W2, b2, dropout):
    return _run(x, adj, edge_mask, W1, b1, W2, b2)
